# baseline (speedup 1.0000x reference)
import functools

import jax
import jax.numpy as jnp
from jax import lax
from jax.experimental import pallas as pl
from jax.experimental.pallas import tpu as pltpu
from jax.experimental.pallas import tpu_sc as plsc

NC = 2
NS = 16
NW = NC * NS


def _sc_segment_sum(src3, dst3, dstf, iota5, p, z_acc, z_deg, with_deg):
    _, NCH, CH = src3.shape
    EC = dstf.shape[1]
    W = p.shape[1]
    n = z_acc.shape[0]
    NR = n // NS
    NH = n // 16

    K = 8
    NB = NCH // K

    out_types = [jax.ShapeDtypeStruct((NC, n, W), jnp.float32)]
    scratch = [
        pltpu.VMEM((NCH, CH), jnp.int32),
        pltpu.VMEM((NCH, CH), jnp.int32),
        pltpu.VMEM((K, CH, W), jnp.float32),
        pltpu.VMEM_SHARED((n, W), jnp.float32),
        pltpu.SemaphoreType.DMA((K,)),
        pltpu.SemaphoreType.DMA((K,)),
    ]
    if with_deg:
        out_types.append(jax.ShapeDtypeStruct((NC, n, 32), jnp.float32))
        scratch += [
            pltpu.VMEM((EC,), jnp.int32),
            pltpu.VMEM((NH // 128, 128), jnp.int32),
            pltpu.VMEM((NH, 16), jnp.float32),
            pltpu.VMEM((NR // 16, 16), jnp.float32),
            pltpu.VMEM((NR, 32), jnp.float32),
            pltpu.VMEM_SHARED((NH, 16), jnp.float32),
        ]

    def body(src_h, dst_h, dstf_h, iota_h, p_h, zacc_h, zdeg_h, agg_o, *rest):
        if with_deg:
            (degp_o, src_v, dst_v, rows_v, acc_sh, gsem, ssem,
             dstf_v, iota_v, hist_v, buf_v, strp_v, deg_sh) = rest
        else:
            src_v, dst_v, rows_v, acc_sh, gsem, ssem = rest
        c = lax.axis_index("c")
        s = lax.axis_index("s")
        wid = s * NC + c
        r0 = s * NR

        pltpu.sync_copy(src_h.at[wid], src_v)
        pltpu.sync_copy(dst_h.at[wid], dst_v)
        pltpu.sync_copy(zacc_h.at[pl.ds(r0, NR)], acc_sh.at[pl.ds(r0, NR)])
        if with_deg:
            pltpu.sync_copy(dstf_h.at[wid], dstf_v)
            pltpu.sync_copy(iota_h, iota_v)
            pltpu.sync_copy(zdeg_h.at[pl.ds(s * (NH // NS), NH // NS)],
                            deg_sh.at[pl.ds(s * (NH // NS), NH // NS)])
            zeros16 = jnp.zeros((16,), jnp.float32)
            ones16 = jnp.ones((16,), jnp.float32)

            def zloop(i, carry):
                hist_v[i] = zeros16
                return carry

            lax.fori_loop(0, NH, zloop, 0)

            def hloop(i, carry):
                d = dstf_v[pl.ds(i * 16, 16)]
                hi = lax.shift_right_logical(d, 4)
                lo = lax.bitwise_and(d, 15)
                plsc.addupdate_scatter(hist_v, [hi, lo], ones16)
                return carry

            lax.fori_loop(0, EC // 16, hloop, 0)
        plsc.subcore_barrier()

        if with_deg:
            mds = [pltpu.async_copy(hist_v.at[pl.ds(q * 128, 128)],
                                    deg_sh.at[iota_v.at[q]],
                                    gsem.at[q % K], add=True)
                   for q in range(NH // 128)]

        def pipe(b, carry):
            gds = [pltpu.async_copy(p_h.at[src_v.at[b * K + k]],
                                    rows_v.at[k], gsem.at[k])
                   for k in range(K)]
            sds = []
            for k in range(K):
                gds[k].wait()
                j = b * K + k
                sds.append(pltpu.async_copy(rows_v.at[k],
                                            acc_sh.at[dst_v.at[j]],
                                            ssem.at[k], add=True))
            for d in sds:
                d.wait()
            return carry

        if with_deg:
            for d in mds:
                d.wait()
        lax.fori_loop(0, NB, pipe, 0)
        plsc.subcore_barrier()

        pltpu.sync_copy(acc_sh.at[pl.ds(r0, NR)], agg_o.at[c, pl.ds(r0, NR)])
        if with_deg:
            pltpu.sync_copy(deg_sh.at[pl.ds(r0 // 16, NR // 16)], buf_v)

            def rloop(k, carry):
                d16 = buf_v[k]
                rows = k * 16 + lax.iota(jnp.int32, 16)
                for col in range(32):
                    plsc.store_scatter(
                        strp_v, [rows, jnp.full((16,), col, jnp.int32)], d16)
                return carry

            lax.fori_loop(0, NR // 16, rloop, 0)
            pltpu.sync_copy(strp_v, degp_o.at[c, pl.ds(r0, NR)])

    f = pl.kernel(
        body,
        out_type=tuple(out_types) if with_deg else out_types[0],
        mesh=plsc.VectorSubcoreMesh(core_axis_name="c", subcore_axis_name="s",
                                    num_cores=NC, num_subcores=NS),
        scratch_types=tuple(scratch),
        compiler_params=pltpu.CompilerParams(use_tc_tiling_on_sc=False,
                                             needs_layout_passes=False),
    )
    return f(src3, dst3, dstf, iota5, p, z_acc, z_deg)


def _dot_t(a, w):
    return lax.dot_general(a, w, (((1,), (1,)), ((), ())),
                           preferred_element_type=jnp.float32)


def _unpack4(a):
    m = a.shape[0]
    parts = [a[:, 32 * j:32 * (j + 1)] for j in range(4)]
    return jnp.stack(parts, axis=1).reshape(4 * m, 32)


def _tc_project(x, Wl, Wr, bl, npad):
    n, d = x.shape
    h = Wl.shape[0]

    def body(x_ref, wl_ref, wr_ref, bl_ref, p_ref, r_ref):
        xb = x_ref[...]
        p_ref[:n, :h] = _dot_t(xb, wl_ref[...])
        r_ref[:n, :] = _dot_t(xb, wr_ref[...]) + bl_ref[...]

    return pl.pallas_call(
        body,
        out_shape=[jax.ShapeDtypeStruct((npad, 128), jnp.float32),
                   jax.ShapeDtypeStruct((npad, h), jnp.float32)],
    )(x, Wl, Wr, bl.reshape(1, h))


def _combine(agg_blk, deg_blk, r):
    a = agg_blk[0] + agg_blk[1]
    dg = deg_blk[0] + deg_blk[1]
    mean = _unpack4(a / jnp.maximum(dg, 1.0))
    out = mean + r
    nrm = jnp.sqrt(jnp.sum(out * out, axis=-1, keepdims=True))
    return out / jnp.maximum(nrm, 1e-12)


def _tc_layer2_in(aggpk, degpk, r1, Wl2p, Wr2p, bl2p, br=1280):
    npad, w = r1.shape

    def body(agg_ref, deg_ref, r1_ref, wl_ref, wr_ref, bl_ref,
             p2_ref, r2_ref):
        h = jnp.maximum(
            _combine(agg_ref[...], deg_ref[...], r1_ref[...]), 0.0)
        p2_ref[:, :w] = _dot_t(h, wl_ref[...])
        r2_ref[...] = _dot_t(h, wr_ref[...]) + bl_ref[...]

    return pl.pallas_call(
        body,
        grid=(npad // br,),
        in_specs=[
            pl.BlockSpec((NC, br // 4, 128), lambda i: (0, i, 0)),
            pl.BlockSpec((NC, br // 4, 128), lambda i: (0, i, 0)),
            pl.BlockSpec((br, w), lambda i: (i, 0)),
            pl.BlockSpec((w, w), lambda i: (0, 0)),
            pl.BlockSpec((w, w), lambda i: (0, 0)),
            pl.BlockSpec((1, w), lambda i: (0, 0)),
        ],
        out_specs=[
            pl.BlockSpec((br, 128), lambda i: (i, 0)),
            pl.BlockSpec((br, w), lambda i: (i, 0)),
        ],
        out_shape=[jax.ShapeDtypeStruct((npad, 128), jnp.float32),
                   jax.ShapeDtypeStruct((npad, w), jnp.float32)],
    )(aggpk, degpk, r1, Wl2p, Wr2p, bl2p)


def _tc_head(aggpk, degpk, r2, Wlinp, blin, br=1280):
    npad, w = r2.shape
    co = Wlinp.shape[0]

    def body(agg_ref, deg_ref, r2_ref, wl_ref, bl_ref, o_ref):
        h2 = _combine(agg_ref[...], deg_ref[...], r2_ref[...])
        logits = _dot_t(h2, wl_ref[...]) + bl_ref[...]
        m = jnp.max(logits, axis=-1, keepdims=True)
        e = jnp.exp(logits - m)
        o_ref[...] = e / jnp.sum(e, axis=-1, keepdims=True)

    return pl.pallas_call(
        body,
        grid=(npad // br,),
        in_specs=[
            pl.BlockSpec((NC, br // 4, 128), lambda i: (0, i, 0)),
            pl.BlockSpec((NC, br // 4, 128), lambda i: (0, i, 0)),
            pl.BlockSpec((br, w), lambda i: (i, 0)),
            pl.BlockSpec((co, w), lambda i: (0, 0)),
            pl.BlockSpec((1, co), lambda i: (0, 0)),
        ],
        out_specs=pl.BlockSpec((br, co), lambda i: (i, 0)),
        out_shape=jax.ShapeDtypeStruct((npad, co), jnp.float32),
    )(aggpk, degpk, r2, Wlinp, blin.reshape(1, co))


def kernel(x, edge_index, Wl1, bl1, Wr1, Wl2, bl2, Wr2, Wlin, blin):
    n, d = x.shape
    e = edge_index.shape[1]
    h1 = Wl1.shape[0]
    h2 = Wl2.shape[0]

    ec = e // NW
    ch = 125
    nch = ec // ch
    src3 = (edge_index[0] * 4).reshape(NW, nch, ch)
    dst3 = edge_index[1].reshape(NW, nch, ch)
    dstf = edge_index[1].reshape(NW, ec)

    npad = ((n + 1023) // 1024) * 1024
    iota5 = jnp.arange(npad // 16, dtype=jnp.int32).reshape(-1, 128)
    z_acc = jnp.zeros((npad, h1), jnp.float32)
    z_deg = jnp.zeros((npad // 16, 16), jnp.float32)

    Wl2p = jnp.pad(Wl2, ((0, h1 - h2), (0, 0)))
    Wr2p = jnp.pad(Wr2, ((0, h1 - h2), (0, 0)))
    bl2p = jnp.pad(bl2, (0, h1 - h2)).reshape(1, h1)
    Wlinp = jnp.pad(Wlin, ((0, 0), (0, h1 - h2)))

    p1t, r1 = _tc_project(x, Wl1, Wr1, bl1, npad)
    aggp1, degp = _sc_segment_sum(src3, dst3, dstf, iota5,
                                  p1t.reshape(4 * npad, h1),
                                  z_acc, z_deg, True)
    aggp1k = aggp1.reshape(NC, npad // 4, 128)
    degpk = degp.reshape(NC, npad // 4, 128)
    p2t, r2 = _tc_layer2_in(aggp1k, degpk, r1, Wl2p, Wr2p, bl2p)
    aggp2 = _sc_segment_sum(src3, dst3, dstf, iota5,
                            p2t.reshape(4 * npad, h1),
                            z_acc, z_deg, False)
    out = _tc_head(aggp2.reshape(NC, npad // 4, 128), degpk, r2, Wlinp, blin)
    return out[:n]


# speedup vs baseline: 1.0125x; 1.0125x over previous
import functools

import jax
import jax.numpy as jnp
from jax import lax
from jax.experimental import pallas as pl
from jax.experimental.pallas import tpu as pltpu
from jax.experimental.pallas import tpu_sc as plsc

NC = 2
NS = 16
NW = NC * NS


def _sc_segment_sum(src3, dst3, p, z_acc, z_deg, ones_b, with_deg):
    _, NCH, CH = src3.shape
    W = p.shape[1]
    n = z_acc.shape[0]
    NR = n // NS

    K = 8
    NB = NCH // K

    out_types = [jax.ShapeDtypeStruct((NC, n, W), jnp.float32)]
    scratch = [
        pltpu.VMEM((NCH, CH), jnp.int32),
        pltpu.VMEM((NCH, CH), jnp.int32),
        pltpu.VMEM((K, CH, W), jnp.float32),
        pltpu.VMEM_SHARED((n, W), jnp.float32),
        pltpu.SemaphoreType.DMA((K,)),
        pltpu.SemaphoreType.DMA((K,)),
    ]
    if with_deg:
        out_types.append(jax.ShapeDtypeStruct((NC, n, 32), jnp.float32))
        scratch += [
            pltpu.VMEM((CH, 32), jnp.float32),
            pltpu.VMEM_SHARED((n, 32), jnp.float32),
        ]

    def body(src_h, dst_h, p_h, zacc_h, zdeg_h, ones_h, agg_o, *rest):
        if with_deg:
            degp_o, src_v, dst_v, rows_v, acc_sh, gsem, ssem, ones_v, deg_sh = rest
        else:
            src_v, dst_v, rows_v, acc_sh, gsem, ssem = rest
        c = lax.axis_index("c")
        s = lax.axis_index("s")
        wid = s * NC + c
        r0 = s * NR

        pltpu.sync_copy(src_h.at[wid], src_v)
        pltpu.sync_copy(dst_h.at[wid], dst_v)
        pltpu.sync_copy(zacc_h.at[pl.ds(r0, NR)], acc_sh.at[pl.ds(r0, NR)])
        if with_deg:
            pltpu.sync_copy(ones_h, ones_v)
            pltpu.sync_copy(zdeg_h.at[pl.ds(r0, NR)], deg_sh.at[pl.ds(r0, NR)])
        plsc.subcore_barrier()

        def pipe(b, carry):
            gds = [pltpu.async_copy(p_h.at[src_v.at[b * K + k]],
                                    rows_v.at[k], gsem.at[k])
                   for k in range(K)]
            sds = []
            for k in range(K):
                gds[k].wait()
                j = b * K + k
                sds.append(pltpu.async_copy(rows_v.at[k],
                                            acc_sh.at[dst_v.at[j]],
                                            ssem.at[k], add=True))
                if with_deg:
                    sds.append(pltpu.async_copy(ones_v, deg_sh.at[dst_v.at[j]],
                                                ssem.at[k], add=True))
            for d in sds:
                d.wait()
            return carry

        lax.fori_loop(0, NB, pipe, 0)
        plsc.subcore_barrier()

        pltpu.sync_copy(acc_sh.at[pl.ds(r0, NR)], agg_o.at[c, pl.ds(r0, NR)])
        if with_deg:
            pltpu.sync_copy(deg_sh.at[pl.ds(r0, NR)],
                            degp_o.at[c, pl.ds(r0, NR)])

    f = pl.kernel(
        body,
        out_type=tuple(out_types) if with_deg else out_types[0],
        mesh=plsc.VectorSubcoreMesh(core_axis_name="c", subcore_axis_name="s",
                                    num_cores=NC, num_subcores=NS),
        scratch_types=tuple(scratch),
        compiler_params=pltpu.CompilerParams(use_tc_tiling_on_sc=False),
    )
    return f(src3, dst3, p, z_acc, z_deg, ones_b)


def _dot_t(a, w):
    return lax.dot_general(a, w, (((1,), (1,)), ((), ())),
                           preferred_element_type=jnp.float32)


def _unpack4(a):
    m = a.shape[0]
    parts = [a[:, 32 * j:32 * (j + 1)] for j in range(4)]
    return jnp.stack(parts, axis=1).reshape(4 * m, 32)


def _tc_project(x, Wl, Wr, bl, npad):
    n, d = x.shape
    h = Wl.shape[0]

    def body(x_ref, wl_ref, wr_ref, bl_ref, p_ref, r_ref):
        xb = x_ref[...]
        p_ref[:n, :h] = _dot_t(xb, wl_ref[...])
        r_ref[:n, :] = _dot_t(xb, wr_ref[...]) + bl_ref[...]

    return pl.pallas_call(
        body,
        out_shape=[jax.ShapeDtypeStruct((npad, 128), jnp.float32),
                   jax.ShapeDtypeStruct((npad, h), jnp.float32)],
    )(x, Wl, Wr, bl.reshape(1, h))


def _combine(agg_blk, deg_blk, r):
    a = agg_blk[0] + agg_blk[1]
    dg = deg_blk[0] + deg_blk[1]
    mean = _unpack4(a / jnp.maximum(dg, 1.0))
    out = mean + r
    nrm = jnp.sqrt(jnp.sum(out * out, axis=-1, keepdims=True))
    return out / jnp.maximum(nrm, 1e-12)


def _tc_layer2_in(aggpk, degpk, r1, Wl2p, Wr2p, bl2p, br=1280):
    npad, w = r1.shape

    def body(agg_ref, deg_ref, r1_ref, wl_ref, wr_ref, bl_ref,
             p2_ref, r2_ref):
        h = jnp.maximum(
            _combine(agg_ref[...], deg_ref[...], r1_ref[...]), 0.0)
        p2_ref[:, :w] = _dot_t(h, wl_ref[...])
        r2_ref[...] = _dot_t(h, wr_ref[...]) + bl_ref[...]

    return pl.pallas_call(
        body,
        grid=(npad // br,),
        in_specs=[
            pl.BlockSpec((NC, br // 4, 128), lambda i: (0, i, 0)),
            pl.BlockSpec((NC, br // 4, 128), lambda i: (0, i, 0)),
            pl.BlockSpec((br, w), lambda i: (i, 0)),
            pl.BlockSpec((w, w), lambda i: (0, 0)),
            pl.BlockSpec((w, w), lambda i: (0, 0)),
            pl.BlockSpec((1, w), lambda i: (0, 0)),
        ],
        out_specs=[
            pl.BlockSpec((br, 128), lambda i: (i, 0)),
            pl.BlockSpec((br, w), lambda i: (i, 0)),
        ],
        out_shape=[jax.ShapeDtypeStruct((npad, 128), jnp.float32),
                   jax.ShapeDtypeStruct((npad, w), jnp.float32)],
    )(aggpk, degpk, r1, Wl2p, Wr2p, bl2p)


def _tc_head(aggpk, degpk, r2, Wlinp, blin, br=1280):
    npad, w = r2.shape
    co = Wlinp.shape[0]

    def body(agg_ref, deg_ref, r2_ref, wl_ref, bl_ref, o_ref):
        h2 = _combine(agg_ref[...], deg_ref[...], r2_ref[...])
        logits = _dot_t(h2, wl_ref[...]) + bl_ref[...]
        m = jnp.max(logits, axis=-1, keepdims=True)
        e = jnp.exp(logits - m)
        o_ref[...] = e / jnp.sum(e, axis=-1, keepdims=True)

    return pl.pallas_call(
        body,
        grid=(npad // br,),
        in_specs=[
            pl.BlockSpec((NC, br // 4, 128), lambda i: (0, i, 0)),
            pl.BlockSpec((NC, br // 4, 128), lambda i: (0, i, 0)),
            pl.BlockSpec((br, w), lambda i: (i, 0)),
            pl.BlockSpec((co, w), lambda i: (0, 0)),
            pl.BlockSpec((1, co), lambda i: (0, 0)),
        ],
        out_specs=pl.BlockSpec((br, co), lambda i: (i, 0)),
        out_shape=jax.ShapeDtypeStruct((npad, co), jnp.float32),
    )(aggpk, degpk, r2, Wlinp, blin.reshape(1, co))


def kernel(x, edge_index, Wl1, bl1, Wr1, Wl2, bl2, Wr2, Wlin, blin):
    n, d = x.shape
    e = edge_index.shape[1]
    h1 = Wl1.shape[0]
    h2 = Wl2.shape[0]

    ec = e // NW
    ch = 125
    nch = ec // ch
    src3 = (edge_index[0] * 4).reshape(NW, nch, ch)
    dst3 = edge_index[1].reshape(NW, nch, ch)

    npad = ((n + 1023) // 1024) * 1024
    z_acc = jnp.zeros((npad, h1), jnp.float32)
    z_deg = jnp.zeros((npad, 32), jnp.float32)
    ones_b = jnp.ones((ch, 32), jnp.float32)

    Wl2p = jnp.pad(Wl2, ((0, h1 - h2), (0, 0)))
    Wr2p = jnp.pad(Wr2, ((0, h1 - h2), (0, 0)))
    bl2p = jnp.pad(bl2, (0, h1 - h2)).reshape(1, h1)
    Wlinp = jnp.pad(Wlin, ((0, 0), (0, h1 - h2)))

    p1t, r1 = _tc_project(x, Wl1, Wr1, bl1, npad)
    aggp1, degp = _sc_segment_sum(src3, dst3, p1t.reshape(4 * npad, h1),
                                  z_acc, z_deg, ones_b, True)
    aggp1k = aggp1.reshape(NC, npad // 4, 128)
    degpk = degp.reshape(NC, npad // 4, 128)
    p2t, r2 = _tc_layer2_in(aggp1k, degpk, r1, Wl2p, Wr2p, bl2p)
    aggp2 = _sc_segment_sum(src3, dst3, p2t.reshape(4 * npad, h1),
                            z_acc, z_deg, ones_b, False)
    out = _tc_head(aggp2.reshape(NC, npad // 4, 128), degpk, r2, Wlinp, blin)
    return out[:n]
